# Initial kernel scaffold; baseline (speedup 1.0000x reference)
import jax
import jax.numpy as jnp
from jax.experimental import pallas as pl


def kernel(x_stroke, x_loop, params, edge_index_represents, edge_index_represented_by, edge_index_neighboring_vertical, edge_index_neighboring_horizontal, edge_index_contains, edge_index_order, edge_index_perpendicular):
    raise NotImplementedError("write your pallas kernel here")


# speedup vs baseline: 1.0396x; 1.0396x over previous
import functools
import jax
import jax.numpy as jnp
from jax.experimental import pallas as pl

_RELS = [
    ('represents', 'stroke', 'loop', 'sum'),
    ('represented_by', 'loop', 'stroke', 'sum'),
    ('neighboring_vertical', 'stroke', 'stroke', 'mean'),
    ('neighboring_horizontal', 'stroke', 'stroke', 'mean'),
    ('contains', 'loop', 'loop', 'sum'),
    ('order', 'stroke', 'stroke', 'sum'),
    ('perpendicular', 'stroke', 'stroke', 'mean'),
]
_NODE_TYPES = ('stroke', 'loop')


def _mm_body_res(x_ref, w_ref, b_ref, r_ref, o_ref, *, relu):
    acc = jnp.dot(x_ref[...], w_ref[...], preferred_element_type=jnp.float32)
    acc = acc + b_ref[...] + r_ref[...]
    if relu:
        acc = jnp.maximum(acc, 0.0)
    o_ref[...] = acc


def _mm_body(x_ref, w_ref, b_ref, o_ref, *, relu):
    acc = jnp.dot(x_ref[...], w_ref[...], preferred_element_type=jnp.float32)
    acc = acc + b_ref[...]
    if relu:
        acc = jnp.maximum(acc, 0.0)
    o_ref[...] = acc


def _mm(xstack, w, b, res, relu):
    n, din = xstack.shape
    dout = w.shape[1]
    bn = 1000
    grid = (n // bn,)
    in_specs = [
        pl.BlockSpec((bn, din), lambda i: (i, 0)),
        pl.BlockSpec((din, dout), lambda i: (0, 0)),
        pl.BlockSpec((1, dout), lambda i: (0, 0)),
    ]
    args = [xstack, w, b.reshape(1, dout)]
    if res is not None:
        in_specs.append(pl.BlockSpec((bn, dout), lambda i: (i, 0)))
        args.append(res)
        body = functools.partial(_mm_body_res, relu=relu)
    else:
        body = functools.partial(_mm_body, relu=relu)
    return pl.pallas_call(
        body,
        grid=grid,
        in_specs=in_specs,
        out_specs=pl.BlockSpec((bn, dout), lambda i: (i, 0)),
        out_shape=jax.ShapeDtypeStruct((n, dout), jnp.float32),
    )(*args)


def kernel(x_stroke, x_loop, params,
           edge_index_represents, edge_index_represented_by,
           edge_index_neighboring_vertical, edge_index_neighboring_horizontal,
           edge_index_contains, edge_index_order, edge_index_perpendicular):
    ed = {
        'represents': edge_index_represents,
        'represented_by': edge_index_represented_by,
        'neighboring_vertical': edge_index_neighboring_vertical,
        'neighboring_horizontal': edge_index_neighboring_horizontal,
        'contains': edge_index_contains,
        'order': edge_index_order,
        'perpendicular': edge_index_perpendicular,
    }
    n = {'stroke': x_stroke.shape[0], 'loop': x_loop.shape[0]}

    inv = {}
    for name, src, dst, agg in _RELS:
        if agg == 'mean':
            cnt = jax.ops.segment_sum(
                jnp.ones((ed[name].shape[1],), jnp.float32), ed[name][1],
                num_segments=n[dst])
            inv[name] = 1.0 / jnp.maximum(cnt, 1.0)

    xd = {'stroke': x_stroke, 'loop': x_loop}
    for li, lp in enumerate(params):
        y = {}
        for name, src, dst, agg in _RELS:
            s = jax.ops.segment_sum(
                jnp.take(xd[src], ed[name][0], axis=0), ed[name][1],
                num_segments=n[dst])
            if agg == 'mean':
                s = s * inv[name][:, None]
            y[name] = s
        h = {}
        for nt in _NODE_TYPES:
            w_root = lp['W_root_' + nt]
            has_res_w = ('W_res_' + nt) in lp
            if li > 0 and has_res_w:
                w_root = w_root + lp['W_res_' + nt]
            parts_x = [xd[nt]]
            parts_w = [w_root]
            for name, src, dst, agg in _RELS:
                if dst == nt:
                    parts_x.append(y[name])
                    parts_w.append(lp['W_' + name])
            xstack = jnp.concatenate(parts_x, axis=1)
            wstack = jnp.concatenate(parts_w, axis=0)
            res = xd[nt] if (li > 0 and not has_res_w) else None
            h[nt] = _mm(xstack, wstack, lp['b_' + nt], res, relu=(li > 0))
        xd = h
    return (xd['stroke'], xd['loop'])


# speedup vs baseline: 1.4168x; 1.3628x over previous
import functools
import jax
import jax.numpy as jnp
from jax import lax
from jax.experimental import pallas as pl
from jax.experimental.pallas import tpu as pltpu
from jax.experimental.pallas import tpu_sc as plsc

_RELS = [
    ('represents', 'stroke', 'loop', 'sum'),
    ('represented_by', 'loop', 'stroke', 'sum'),
    ('neighboring_vertical', 'stroke', 'stroke', 'mean'),
    ('neighboring_horizontal', 'stroke', 'stroke', 'mean'),
    ('contains', 'loop', 'loop', 'sum'),
    ('order', 'stroke', 'stroke', 'sum'),
    ('perpendicular', 'stroke', 'stroke', 'mean'),
]
_TBL_ORDER = ['represents', 'contains', 'represented_by', 'order',
              'neighboring_vertical', 'neighboring_horizontal', 'perpendicular']
_NODE_TYPES = ('stroke', 'loop')
_RBY = {name: (src, dst, agg) for name, src, dst, agg in _RELS}
_DINP = 128
_R = 512


def _ceil_to(x, m):
    return ((x + m - 1) // m) * m


def _seg_sum_sc(xcat, src_s, dst_s, offs, n_pad, dinp, r):
    chunks = n_pad // r
    sweeps = -(-chunks // 32)
    info = plsc.get_sparse_core_info()
    nc = info.num_cores
    un = dinp // 16
    offlen = offs.shape[0]

    def body(x_hbm, src_hbm, dst_hbm, off_hbm, y_hbm,
             off_v, sidx_v, didx_v, rows_v, acc_v, sem):
        wid = lax.axis_index("s") * nc + lax.axis_index("c")
        pltpu.sync_copy(off_hbm, off_v)

        def do_chunk(c):
            base = c * r

            def zr(row, carry):
                for u in range(un):
                    acc_v[row, pl.ds(u * 16, 16)] = jnp.zeros((16,), jnp.float32)
                return carry
            lax.fori_loop(0, r + 1, zr, 0)

            ovec = off_v[pl.ds(c, 16)]
            e0 = ovec[0]
            e1 = ovec[1]
            ea = (e0 // 128) * 128
            nb = (e1 - ea + 127) // 128

            def blk(j, carry):
                e = pl.multiple_of(ea + j * 128, 128)
                pltpu.sync_copy(src_hbm.at[pl.ds(e, 128)], sidx_v)
                pltpu.sync_copy(dst_hbm.at[pl.ds(e, 128)], didx_v)
                pltpu.async_copy(x_hbm.at[sidx_v], rows_v, sem).wait()

                def ed16(kb, icarry):
                    dvec = didx_v[pl.ds(kb * 16, 16)] - base
                    for i in range(16):
                        local = dvec[i]
                        tgt = jnp.where((local >= 0) & (local < r), local, r)
                        k = kb * 16 + i

                        def uu(u, uc):
                            sl = pl.ds(u * 16, 16)
                            acc_v[tgt, sl] = acc_v[tgt, sl] + rows_v[k, sl]
                            return uc
                        lax.fori_loop(0, un, uu, 0)
                    return icarry
                lax.fori_loop(0, 8, ed16, 0)
                return carry
            lax.fori_loop(0, nb, blk, 0)
            pltpu.sync_copy(acc_v.at[pl.ds(0, r)], y_hbm.at[pl.ds(base, r)])

        def sweep(cc, carry):
            c = cc * 32 + wid

            @pl.when(c < chunks)
            def _():
                do_chunk(c)
            return carry
        lax.fori_loop(0, sweeps, sweep, 0)

    f = pl.kernel(
        body,
        out_type=jax.ShapeDtypeStruct((n_pad, dinp), jnp.float32),
        mesh=plsc.VectorSubcoreMesh(core_axis_name="c", subcore_axis_name="s"),
        scratch_types=[
            pltpu.VMEM((offlen,), jnp.int32),
            pltpu.VMEM((128,), jnp.int32),
            pltpu.VMEM((128,), jnp.int32),
            pltpu.VMEM((128, dinp), jnp.float32),
            pltpu.VMEM((r + 1, dinp), jnp.float32),
            pltpu.SemaphoreType.DMA,
        ],
    )
    return f(xcat, src_s, dst_s, offs)


def _tc_body(*refs, nslice, mean_flags, relu, ident_res):
    x_ref = refs[0]
    y_refs = refs[1:1 + nslice]
    ncnt = sum(mean_flags)
    cnt_refs = refs[1 + nslice:1 + nslice + ncnt]
    w_ref, b_ref, o_ref = refs[1 + nslice + ncnt:]
    parts = [x_ref[...]]
    ci = 0
    for j in range(nslice):
        yj = y_refs[j][...]
        if mean_flags[j]:
            inv = 1.0 / jnp.maximum(cnt_refs[ci][...], 1.0)
            yj = yj * inv
            ci += 1
        parts.append(yj)
    xcat = jnp.concatenate(parts, axis=1)
    acc = jnp.dot(xcat, w_ref[...], preferred_element_type=jnp.float32)
    acc = acc + b_ref[...]
    if ident_res:
        acc = acc + x_ref[...]
    if relu:
        acc = jnp.maximum(acc, 0.0)
    o_ref[...] = acc


def _tc_layer(xcat, ytab, cnt, wstack, bias, xoff, n, slice_offs, mean_flags,
              relu, ident_res, dinp, dout):
    bn = 1000
    nslice = len(slice_offs)
    in_specs = [pl.BlockSpec((bn, dinp), lambda i, o=xoff // bn: (o + i, 0))]
    args = [xcat]
    for off in slice_offs:
        in_specs.append(pl.BlockSpec((bn, dinp), lambda i, o=off // bn: (o + i, 0)))
        args.append(ytab)
    for j, off in enumerate(slice_offs):
        if mean_flags[j]:
            in_specs.append(pl.BlockSpec((bn, 1), lambda i, o=off // bn: (o + i, 0)))
            args.append(cnt)
    in_specs.append(pl.BlockSpec(((1 + nslice) * dinp, dout), lambda i: (0, 0)))
    args.append(wstack)
    in_specs.append(pl.BlockSpec((1, dout), lambda i: (0, 0)))
    args.append(bias.reshape(1, dout))
    body = functools.partial(_tc_body, nslice=nslice, mean_flags=mean_flags,
                             relu=relu, ident_res=ident_res)
    return pl.pallas_call(
        body,
        grid=(n // bn,),
        in_specs=in_specs,
        out_specs=pl.BlockSpec((bn, dout), lambda i: (i, 0)),
        out_shape=jax.ShapeDtypeStruct((n, dout), jnp.float32),
    )(*args)


def kernel(x_stroke, x_loop, params,
           edge_index_represents, edge_index_represented_by,
           edge_index_neighboring_vertical, edge_index_neighboring_horizontal,
           edge_index_contains, edge_index_order, edge_index_perpendicular):
    ed = {
        'represents': edge_index_represents,
        'represented_by': edge_index_represented_by,
        'neighboring_vertical': edge_index_neighboring_vertical,
        'neighboring_horizontal': edge_index_neighboring_horizontal,
        'contains': edge_index_contains,
        'order': edge_index_order,
        'perpendicular': edge_index_perpendicular,
    }
    n = {'stroke': x_stroke.shape[0], 'loop': x_loop.shape[0]}

    tbl_off = {}
    pos = 0
    for name in _TBL_ORDER:
        tbl_off[name] = pos
        pos += n[_RBY[name][1]]
    n_pad = _ceil_to(pos, _R)

    src_parts, dst_parts = [], []
    for name in _TBL_ORDER:
        src_t, dst_t = _RBY[name][0], _RBY[name][1]
        soff = 0 if src_t == 'stroke' else n['stroke']
        src_parts.append(ed[name][0] + soff)
        dst_parts.append(ed[name][1] + tbl_off[name])
    src_g = jnp.concatenate(src_parts)
    dst_g = jnp.concatenate(dst_parts)
    e_tot = src_g.shape[0]
    e_pad = _ceil_to(e_tot, 128)
    dst_s, src_s = lax.sort_key_val(dst_g, src_g)
    src_s = jnp.pad(src_s, (0, e_pad - e_tot))
    dst_s = jnp.pad(dst_s, (0, e_pad - e_tot), constant_values=n_pad - 1)

    chunks = n_pad // _R
    o = jnp.searchsorted(dst_s, jnp.arange(0, n_pad + 1, _R, dtype=jnp.int32),
                         side='left').astype(jnp.int32)
    offs = jnp.pad(o, (0, _ceil_to(chunks + 17, 16) - (chunks + 1)),
                   constant_values=e_pad)

    def pad0(x):
        m = x.shape[0]
        return jnp.concatenate(
            [x, jnp.ones((m, 1), jnp.float32),
             jnp.zeros((m, _DINP - 10), jnp.float32)], axis=1)
    xcat = jnp.concatenate([pad0(x_stroke), pad0(x_loop)], axis=0)
    cnt = None

    slices_nt = {
        'stroke': ['represented_by', 'order', 'neighboring_vertical',
                   'neighboring_horizontal', 'perpendicular'],
        'loop': ['represents', 'contains'],
    }

    for li, lp in enumerate(params):
        din = lp['W_root_stroke'].shape[0]
        dout = lp['W_root_stroke'].shape[1]

        ytab = _seg_sum_sc(xcat, src_s, dst_s, offs, n_pad, _DINP, _R)
        if li == 0:
            cnt = lax.slice(ytab, (0, 9), (n_pad, 10))

        outs = {}
        for nt in _NODE_TYPES:
            w_root = lp['W_root_' + nt]
            has_res_w = ('W_res_' + nt) in lp
            if li > 0 and has_res_w:
                w_root = w_root + lp['W_res_' + nt]
            ws = [w_root] + [lp['W_' + name] for name in slices_nt[nt]]
            if din < _DINP:
                ws = [jnp.pad(w, ((0, _DINP - din), (0, 0)))
                      for w in ws]
            wstack = jnp.concatenate(ws, axis=0)
            outs[nt] = _tc_layer(
                xcat, ytab, cnt, wstack, lp['b_' + nt],
                xoff=0 if nt == 'stroke' else n['stroke'],
                n=n[nt],
                slice_offs=[tbl_off[name] for name in slices_nt[nt]],
                mean_flags=[_RBY[name][2] == 'mean' for name in slices_nt[nt]],
                relu=(li > 0),
                ident_res=(li > 0 and not has_res_w),
                dinp=_DINP, dout=dout)
        if dout < _DINP:
            outs_p = {nt: jnp.pad(outs[nt], ((0, 0), (0, _DINP - dout)))
                      for nt in _NODE_TYPES}
        else:
            outs_p = outs
        xcat = jnp.concatenate([outs_p['stroke'], outs_p['loop']], axis=0)

    return (outs['stroke'], outs['loop'])
